# baseline (speedup 1.0000x reference)
import jax
import jax.numpy as jnp
from jax import lax
from jax.experimental import pallas as pl
from jax.experimental.pallas import tpu as pltpu
from jax.experimental.pallas import tpu_sc as plsc

N = 10000
E = 160000
D = 256
HALF = 128
NLAYERS = 3

NC = 2
NS = 16
NW = NC * NS
L = 16

NROWS = 10240
CK = 50
NCHUNK = E // (NW * CK)
NBUF = 5
RPT = NROWS // NS
NB = 16
NZ = RPT // NB
DEPTH = 12


def _counts_body(dst3, recip_out, counts, dst_v, ones_t, rec_t, ssem):
    c = lax.axis_index("c")
    s = lax.axis_index("s")

    @pl.loop(0, NB)
    def _fill_rec(i):
        rec_t[i, :] = jnp.zeros((L,), jnp.float32)

    @pl.loop(0, CK)
    def _fill_ones(i):
        ones_t[i, :] = jnp.ones((L,), jnp.float32)

    @pl.loop(0, NZ)
    def _zero_counts(i):
        pltpu.sync_copy(rec_t, counts.at[pl.ds(s * RPT + i * NB, NB)])

    pltpu.sync_copy(dst3.at[2 * s], dst_v.at[pl.ds(0, NCHUNK)])
    pltpu.sync_copy(dst3.at[2 * s + 1], dst_v.at[pl.ds(NCHUNK, NCHUNK)])
    plsc.subcore_barrier()

    def _scat(q):
        return pltpu.make_async_copy(ones_t, counts.at[dst_v.at[q]], ssem)

    for q in range(DEPTH):
        pltpu.async_copy(ones_t, counts.at[dst_v.at[q]], ssem, add=True)

    @pl.loop(0, 2 * NCHUNK - DEPTH)
    def _count(q):
        _scat(q).wait()
        pltpu.async_copy(ones_t, counts.at[dst_v.at[q + DEPTH]], ssem,
                         add=True)

    @pl.loop(2 * NCHUNK - DEPTH, 2 * NCHUNK)
    def _drain(q):
        _scat(q).wait()

    plsc.subcore_barrier()

    @pl.when(c == 0)
    def _write():
        @pl.loop(0, NZ)
        def _recip(i):
            r0 = s * RPT + i * NB
            pltpu.sync_copy(counts.at[pl.ds(r0, NB)], rec_t)

            @pl.loop(0, NB)
            def _rrow(r):
                rec_t[r, :] = 1.0 / jnp.maximum(rec_t[r, :], 1.0)

            pltpu.sync_copy(rec_t, recip_out.at[pl.ds(r0, NB)])


def _layers_body(x2, srcA, srcB, dst3, recip, out, h1, h2, accum,
                 src_v, dst_v, rows0, rows1, rows2, rows3, rows4,
                 accA, recA, accB, recB, zbuf,
                 gsem, ssem, asem, rsem, wsem, zsem):
    c = lax.axis_index("c")
    s = lax.axis_index("s")
    bufs = (rows0, rows1, rows2, rows3, rows4)

    @pl.loop(0, NB // 2)
    def _fill_z(i):
        for l in range(HALF // L):
            zbuf[i, pl.ds(l * L, L)] = jnp.zeros((L,), jnp.float32)

    for layer in range(NLAYERS):
        if layer == 0:
            with jax.named_scope("ph_zero0"):
                @pl.loop(0, 2 * NZ)
                def _zero_acc(i):
                    pltpu.sync_copy(
                        zbuf, accum.at[pl.ds(s * RPT + i * (NB // 2), NB // 2)])

            plsc.subcore_barrier()

        if layer == 0:
            gsrc = x2.at[pl.ds(c, 2 * N - 1)]
        else:
            gsrc = (h1, h2)[layer - 1].at[c]
        ns_edges = jax.named_scope(f"ph_edges{layer}")
        ns_edges.__enter__()

        def _g(q, b):
            return pltpu.make_async_copy(gsrc.at[src_v.at[q]], bufs[b], gsem)

        def _s(q, b):
            return pltpu.make_async_copy(bufs[b], accum.at[dst_v.at[q]], ssem)

        for half in range(2):
            ns_h = jax.named_scope(f"ph_e{layer}h{half}")
            ns_h.__enter__()
            slab = s + NS * half
            pltpu.sync_copy((srcA if layer == 0 else srcB).at[slab], src_v)
            pltpu.sync_copy(dst3.at[slab], dst_v)

            for b in range(NBUF - 1):
                _g(b, b).start()

            @pl.loop(0, NCHUNK, step=NBUF)
            def _slots(j):
                for b in range(NBUF):
                    q = j + b
                    nb = (b + NBUF - 1) % NBUF
                    _g(q, b).wait()
                    pltpu.async_copy(bufs[b], accum.at[dst_v.at[q]], ssem,
                                     add=True)

                    @pl.when(q + NBUF - 1 < NCHUNK)
                    def _next():
                        @pl.when(q >= 1)
                        def _free():
                            _s(q - 1, nb).wait()

                        _g(q + NBUF - 1, nb).start()

            for k in range(NBUF):
                q = NCHUNK - NBUF + k
                _s(q, q % NBUF).wait()
            ns_h.__exit__(None, None, None)

        ns_edges.__exit__(None, None, None)
        plsc.subcore_barrier()

        ns_norm = jax.named_scope(f"ph_norm{layer}")
        ns_norm.__enter__()
        nbufs = ((accA, recA), (accB, recB))

        def _r0(i):
            return s * RPT + i * NB

        def _lda(i, b):
            return pltpu.make_async_copy(
                accum.at[pl.ds(_r0(i), NB)], nbufs[b][0], asem)

        def _ldr(i, b):
            return pltpu.make_async_copy(
                recip.at[pl.ds(_r0(i), NB)], nbufs[b][1], rsem)

        def _z(i, hh):
            return pltpu.make_async_copy(
                zbuf, accum.at[pl.ds(_r0(i) + hh * (NB // 2), NB // 2)], zsem)

        def _w(i, b):
            hdst = (h1, h2)[layer]
            return pltpu.make_async_copy(
                nbufs[b][0], hdst.at[c, pl.ds(_r0(i), NB)], wsem)

        _lda(0, 0).start()
        _ldr(0, 0).start()

        @pl.loop(0, NZ, step=2)
        def _norm(j):
            for b in range(2):
                i = j + b
                acc_c, rec_c = nbufs[b]
                _lda(i, b).wait()
                _ldr(i, b).wait()

                @pl.when(i + 1 < NZ)
                def _pref():
                    if layer < NLAYERS - 1:
                        @pl.when(i >= 1)
                        def _free():
                            _w(i - 1, 1 - b).wait()

                    _lda(i + 1, 1 - b).start()
                    _ldr(i + 1, 1 - b).start()

                if layer < NLAYERS - 1:
                    _z(i, 0).start()
                    _z(i, 1).start()

                @pl.loop(0, NB, unroll=4)
                def _nrow(r):
                    rv = rec_c[r, :]
                    for l in range(HALF // L):
                        acc_c[r, pl.ds(l * L, L)] = (
                            acc_c[r, pl.ds(l * L, L)] * rv)

                if layer < NLAYERS - 1:
                    _w(i, b).start()
                else:
                    r0 = _r0(i)

                    @pl.when(r0 + NB <= N)
                    def _full():
                        pltpu.sync_copy(
                            acc_c,
                            out.at[pl.ds(r0, NB), pl.ds(c * HALF, HALF)])

                    @pl.when(jnp.logical_and(r0 < N, r0 + NB > N))
                    def _part():
                        pltpu.sync_copy(
                            acc_c.at[pl.ds(0, N % NB)],
                            out.at[pl.ds(r0, N % NB), pl.ds(c * HALF, HALF)])

        if layer < NLAYERS - 1:
            _w(NZ - 2, 0).wait()
            _w(NZ - 1, 1).wait()
            for i in range(NZ):
                _z(i, 0).wait()
                _z(i, 1).wait()

        ns_norm.__exit__(None, None, None)
        plsc.subcore_barrier()


def kernel(x, edge_index):
    src = edge_index[0].astype(jnp.int32)
    dst = edge_index[1].astype(jnp.int32)

    srcA = (2 * src).reshape(NW, NCHUNK, CK)
    srcB = src.reshape(NW, NCHUNK, CK)
    dst3 = dst.reshape(NW, NCHUNK, CK)
    x2 = x.reshape(2 * N, HALF)

    mesh = plsc.VectorSubcoreMesh(core_axis_name="c", subcore_axis_name="s")
    params = pltpu.CompilerParams(use_tc_tiling_on_sc=False)

    counts_f = pl.kernel(
        _counts_body,
        out_type=jax.ShapeDtypeStruct((NROWS, L), jnp.float32),
        mesh=mesh,
        compiler_params=params,
        scratch_types=[
            pltpu.VMEM_SHARED((NROWS, L), jnp.float32),
            pltpu.VMEM((2 * NCHUNK, CK), jnp.int32),
            pltpu.VMEM((CK, L), jnp.float32),
            pltpu.VMEM((NB, L), jnp.float32),
            pltpu.SemaphoreType.DMA,
        ],
    )
    recip = counts_f(dst3)

    layers_f = pl.kernel(
        _layers_body,
        out_type=jax.ShapeDtypeStruct((N, D), jnp.float32),
        mesh=mesh,
        compiler_params=params,
        scratch_types=[
            pltpu.HBM((NC, NROWS, HALF), jnp.float32),
            pltpu.HBM((NC, NROWS, HALF), jnp.float32),
            pltpu.VMEM_SHARED((NROWS, HALF), jnp.float32),
            pltpu.VMEM((NCHUNK, CK), jnp.int32),
            pltpu.VMEM((NCHUNK, CK), jnp.int32),
            pltpu.VMEM((CK, HALF), jnp.float32),
            pltpu.VMEM((CK, HALF), jnp.float32),
            pltpu.VMEM((CK, HALF), jnp.float32),
            pltpu.VMEM((CK, HALF), jnp.float32),
            pltpu.VMEM((CK, HALF), jnp.float32),
            pltpu.VMEM((NB, HALF), jnp.float32),
            pltpu.VMEM((NB, L), jnp.float32),
            pltpu.VMEM((NB, HALF), jnp.float32),
            pltpu.VMEM((NB, L), jnp.float32),
            pltpu.VMEM((NB // 2, HALF), jnp.float32),
            pltpu.SemaphoreType.DMA,
            pltpu.SemaphoreType.DMA,
            pltpu.SemaphoreType.DMA,
            pltpu.SemaphoreType.DMA,
            pltpu.SemaphoreType.DMA,
            pltpu.SemaphoreType.DMA,
        ],
    )
    return layers_f(x2, srcA, srcB, dst3, recip)


# speedup vs baseline: 1.0589x; 1.0589x over previous
import jax
import jax.numpy as jnp
from jax import lax
from jax.experimental import pallas as pl
from jax.experimental.pallas import tpu as pltpu
from jax.experimental.pallas import tpu_sc as plsc

N = 10000
E = 160000
D = 256
HALF = 128
NLAYERS = 3

NC = 2
NS = 16
NW = NC * NS
L = 16

NROWS = 10240
CK = 50
NCHUNK = E // (NW * CK)
NBUF = 4
RPT = NROWS // NS
NB = 32
NZ = RPT // NB
DEPTH = 12


def _counts_body(dst3, recip_out, counts, dst_v, ones_t, rec_t, ssem):
    c = lax.axis_index("c")
    s = lax.axis_index("s")

    @pl.loop(0, NB)
    def _fill_rec(i):
        rec_t[i, :] = jnp.zeros((L,), jnp.float32)

    @pl.loop(0, CK)
    def _fill_ones(i):
        ones_t[i, :] = jnp.ones((L,), jnp.float32)

    @pl.loop(0, NZ)
    def _zero_counts(i):
        pltpu.sync_copy(rec_t, counts.at[pl.ds(s * RPT + i * NB, NB)])

    pltpu.sync_copy(dst3.at[2 * s], dst_v.at[pl.ds(0, NCHUNK)])
    pltpu.sync_copy(dst3.at[2 * s + 1], dst_v.at[pl.ds(NCHUNK, NCHUNK)])
    plsc.subcore_barrier()

    def _scat(q):
        return pltpu.make_async_copy(ones_t, counts.at[dst_v.at[q]], ssem)

    for q in range(DEPTH):
        pltpu.async_copy(ones_t, counts.at[dst_v.at[q]], ssem, add=True)

    @pl.loop(0, 2 * NCHUNK - DEPTH)
    def _count(q):
        _scat(q).wait()
        pltpu.async_copy(ones_t, counts.at[dst_v.at[q + DEPTH]], ssem,
                         add=True)

    @pl.loop(2 * NCHUNK - DEPTH, 2 * NCHUNK)
    def _drain(q):
        _scat(q).wait()

    plsc.subcore_barrier()

    @pl.when(c == 0)
    def _write():
        @pl.loop(0, NZ)
        def _recip(i):
            r0 = s * RPT + i * NB
            pltpu.sync_copy(counts.at[pl.ds(r0, NB)], rec_t)

            @pl.loop(0, NB)
            def _rrow(r):
                rec_t[r, :] = 1.0 / jnp.maximum(rec_t[r, :], 1.0)

            pltpu.sync_copy(rec_t, recip_out.at[pl.ds(r0, NB)])


def _layers_body(x2, srcA, srcB, dst3, recip, out, h1, h2, accum,
                 src_v, dst_v, rows0, rows1, rows2, rows3,
                 accA, recA, accB, recB, zbuf,
                 gsem, ssem, asem, rsem, wsem, zsem):
    c = lax.axis_index("c")
    s = lax.axis_index("s")
    bufs = (rows0, rows1, rows2, rows3)

    @pl.loop(0, NB // 2)
    def _fill_z(i):
        for l in range(HALF // L):
            zbuf[i, pl.ds(l * L, L)] = jnp.zeros((L,), jnp.float32)

    for layer in range(NLAYERS):
        if layer == 0:
            with jax.named_scope("ph_zero0"):
                @pl.loop(0, 2 * NZ)
                def _zero_acc(i):
                    pltpu.sync_copy(
                        zbuf, accum.at[pl.ds(s * RPT + i * (NB // 2), NB // 2)])

            plsc.subcore_barrier()

        if layer == 0:
            gsrc = x2.at[pl.ds(c, 2 * N - 1)]
        else:
            gsrc = (h1, h2)[layer - 1].at[c]
        ns_edges = jax.named_scope(f"ph_edges{layer}")
        ns_edges.__enter__()

        def _g(q, b):
            return pltpu.make_async_copy(gsrc.at[src_v.at[q]], bufs[b], gsem)

        def _s(q, b):
            return pltpu.make_async_copy(bufs[b], accum.at[dst_v.at[q]], ssem)

        for half in range(2):
            ns_h = jax.named_scope(f"ph_e{layer}h{half}")
            ns_h.__enter__()
            slab = s + NS * half
            pltpu.sync_copy((srcA if layer == 0 else srcB).at[slab], src_v)
            pltpu.sync_copy(dst3.at[slab], dst_v)

            for b in range(NBUF - 1):
                _g(b, b).start()

            @pl.loop(0, NCHUNK, step=NBUF)
            def _slots(j):
                for b in range(NBUF):
                    q = j + b
                    nb = (b + NBUF - 1) % NBUF
                    _g(q, b).wait()
                    pltpu.async_copy(bufs[b], accum.at[dst_v.at[q]], ssem,
                                     add=True)

                    @pl.when(q + NBUF - 1 < NCHUNK)
                    def _next():
                        @pl.when(q >= 1)
                        def _free():
                            _s(q - 1, nb).wait()

                        _g(q + NBUF - 1, nb).start()

            for k in range(NBUF):
                q = NCHUNK - NBUF + k
                _s(q, q % NBUF).wait()
            ns_h.__exit__(None, None, None)

        ns_edges.__exit__(None, None, None)
        plsc.subcore_barrier()

        ns_norm = jax.named_scope(f"ph_norm{layer}")
        ns_norm.__enter__()
        nbufs = ((accA, recA), (accB, recB))

        def _r0(i):
            return s * RPT + i * NB

        def _lda(i, b):
            return pltpu.make_async_copy(
                accum.at[pl.ds(_r0(i), NB)], nbufs[b][0], asem)

        def _ldr(i, b):
            return pltpu.make_async_copy(
                recip.at[pl.ds(_r0(i), NB)], nbufs[b][1], rsem)

        def _z(i, hh):
            return pltpu.make_async_copy(
                zbuf, accum.at[pl.ds(_r0(i) + hh * (NB // 2), NB // 2)], zsem)

        def _w(i, b):
            hdst = (h1, h2)[layer]
            return pltpu.make_async_copy(
                nbufs[b][0], hdst.at[c, pl.ds(_r0(i), NB)], wsem)

        _lda(0, 0).start()
        _ldr(0, 0).start()

        @pl.loop(0, NZ, step=2)
        def _norm(j):
            for b in range(2):
                i = j + b
                acc_c, rec_c = nbufs[b]
                _lda(i, b).wait()
                _ldr(i, b).wait()

                @pl.when(i + 1 < NZ)
                def _pref():
                    if layer < NLAYERS - 1:
                        @pl.when(i >= 1)
                        def _free():
                            _w(i - 1, 1 - b).wait()

                    _lda(i + 1, 1 - b).start()
                    _ldr(i + 1, 1 - b).start()

                if layer < NLAYERS - 1:
                    _z(i, 0).start()
                    _z(i, 1).start()

                @pl.loop(0, NB, unroll=4)
                def _nrow(r):
                    rv = rec_c[r, :]
                    for l in range(HALF // L):
                        acc_c[r, pl.ds(l * L, L)] = (
                            acc_c[r, pl.ds(l * L, L)] * rv)

                if layer < NLAYERS - 1:
                    _w(i, b).start()
                else:
                    r0 = _r0(i)

                    @pl.when(r0 + NB <= N)
                    def _full():
                        pltpu.sync_copy(
                            acc_c,
                            out.at[pl.ds(r0, NB), pl.ds(c * HALF, HALF)])

                    @pl.when(jnp.logical_and(r0 < N, r0 + NB > N))
                    def _part():
                        pltpu.sync_copy(
                            acc_c.at[pl.ds(0, N % NB)],
                            out.at[pl.ds(r0, N % NB), pl.ds(c * HALF, HALF)])

        if layer < NLAYERS - 1:
            _w(NZ - 2, 0).wait()
            _w(NZ - 1, 1).wait()
            for i in range(NZ):
                _z(i, 0).wait()
                _z(i, 1).wait()

        ns_norm.__exit__(None, None, None)
        plsc.subcore_barrier()


def kernel(x, edge_index):
    src = edge_index[0].astype(jnp.int32)
    dst = edge_index[1].astype(jnp.int32)

    srcA = (2 * src).reshape(NW, NCHUNK, CK)
    srcB = src.reshape(NW, NCHUNK, CK)
    dst3 = dst.reshape(NW, NCHUNK, CK)
    x2 = x.reshape(2 * N, HALF)

    mesh = plsc.VectorSubcoreMesh(core_axis_name="c", subcore_axis_name="s")
    params = pltpu.CompilerParams(use_tc_tiling_on_sc=False)

    counts_f = pl.kernel(
        _counts_body,
        out_type=jax.ShapeDtypeStruct((NROWS, L), jnp.float32),
        mesh=mesh,
        compiler_params=params,
        scratch_types=[
            pltpu.VMEM_SHARED((NROWS, L), jnp.float32),
            pltpu.VMEM((2 * NCHUNK, CK), jnp.int32),
            pltpu.VMEM((CK, L), jnp.float32),
            pltpu.VMEM((NB, L), jnp.float32),
            pltpu.SemaphoreType.DMA,
        ],
    )
    recip = counts_f(dst3)

    layers_f = pl.kernel(
        _layers_body,
        out_type=jax.ShapeDtypeStruct((N, D), jnp.float32),
        mesh=mesh,
        compiler_params=params,
        scratch_types=[
            pltpu.HBM((NC, NROWS, HALF), jnp.float32),
            pltpu.HBM((NC, NROWS, HALF), jnp.float32),
            pltpu.VMEM_SHARED((NROWS, HALF), jnp.float32),
            pltpu.VMEM((NCHUNK, CK), jnp.int32),
            pltpu.VMEM((NCHUNK, CK), jnp.int32),
            pltpu.VMEM((CK, HALF), jnp.float32),
            pltpu.VMEM((CK, HALF), jnp.float32),
            pltpu.VMEM((CK, HALF), jnp.float32),
            pltpu.VMEM((CK, HALF), jnp.float32),
            pltpu.VMEM((NB, HALF), jnp.float32),
            pltpu.VMEM((NB, L), jnp.float32),
            pltpu.VMEM((NB, HALF), jnp.float32),
            pltpu.VMEM((NB, L), jnp.float32),
            pltpu.VMEM((NB // 2, HALF), jnp.float32),
            pltpu.SemaphoreType.DMA,
            pltpu.SemaphoreType.DMA,
            pltpu.SemaphoreType.DMA,
            pltpu.SemaphoreType.DMA,
            pltpu.SemaphoreType.DMA,
            pltpu.SemaphoreType.DMA,
        ],
    )
    return layers_f(x2, srcA, srcB, dst3, recip)
